# baseline (device time: 23521 ns/iter reference)
import jax
import jax.numpy as jnp
from jax import lax
from jax.experimental import pallas as pl
from jax.experimental.pallas import tpu as pltpu

N_DEV = 32


def _coords(p):
    z, q = divmod(p, 8)
    y, j = divmod(q, 2)
    return (j ^ (y & 1), y, z)


def _offsets_far_first():
    def avg_hops(o):
        tot = 0
        for j in range(N_DEV):
            a, b = _coords(j), _coords((j + o) % N_DEV)
            tot += sum(abs(u - v) for u, v in zip(a, b))
        return tot / N_DEV

    return sorted(range(1, N_DEV), key=avg_hops, reverse=True)


_SEND_ORDER = _offsets_far_first()


def kernel(x):
    _, m, n = x.shape
    rows = m // N_DEV

    def body(x_ref, out_ref, p1_buf, chunk_ref,
             p1_send, p1_recv, p2_send, p2_recv):
        my = lax.axis_index("i")

        bar = pltpu.get_barrier_semaphore()
        for o in range(1, N_DEV):
            peer = lax.rem(my + o, N_DEV)
            pl.semaphore_signal(
                bar, inc=1,
                device_id=(peer,), device_id_type=pl.DeviceIdType.MESH,
            )
        pl.semaphore_wait(bar, N_DEV - 1)

        p1 = {}
        for o in _SEND_ORDER:
            peer = lax.rem(my + o, N_DEV)
            rdma = pltpu.make_async_remote_copy(
                src_ref=x_ref.at[0, pl.ds(peer * rows, rows)],
                dst_ref=p1_buf.at[o - 1],
                send_sem=p1_send.at[o - 1],
                recv_sem=p1_recv.at[o - 1],
                device_id=(peer,),
                device_id_type=pl.DeviceIdType.MESH,
            )
            rdma.start()
            p1[o] = rdma

        acc = x_ref[0, pl.ds(my * rows, rows), :]
        for o in reversed(_SEND_ORDER):
            p1[o].wait_recv()
            acc = acc + p1_buf[o - 1]
        chunk_ref[...] = acc

        p2 = {}
        for o in _SEND_ORDER:
            peer = lax.rem(my + o, N_DEV)
            rdma = pltpu.make_async_remote_copy(
                src_ref=chunk_ref,
                dst_ref=out_ref.at[pl.ds(my * rows, rows)],
                send_sem=p2_send.at[o - 1],
                recv_sem=p2_recv.at[o - 1],
                device_id=(peer,),
                device_id_type=pl.DeviceIdType.MESH,
            )
            rdma.start()
            p2[o] = rdma
        out_ref[pl.ds(my * rows, rows), :] = acc

        for o in _SEND_ORDER:
            p2[o].wait_recv()
        for o in _SEND_ORDER:
            p1[o].wait_send()
            p2[o].wait_send()

    return pl.pallas_call(
        body,
        out_shape=jax.ShapeDtypeStruct((m, n), jnp.float32),
        in_specs=[pl.BlockSpec(memory_space=pltpu.VMEM)],
        out_specs=pl.BlockSpec(memory_space=pltpu.VMEM),
        scratch_shapes=[
            pltpu.VMEM((N_DEV - 1, rows, n), jnp.float32),
            pltpu.VMEM((rows, n), jnp.float32),
            pltpu.SemaphoreType.DMA((N_DEV - 1,)),
            pltpu.SemaphoreType.DMA((N_DEV - 1,)),
            pltpu.SemaphoreType.DMA((N_DEV - 1,)),
            pltpu.SemaphoreType.DMA((N_DEV - 1,)),
        ],
        compiler_params=pltpu.CompilerParams(collective_id=0),
    )(x)
